# baseline (speedup 1.0000x reference)
import functools

import jax
import jax.numpy as jnp
from jax import lax
from jax.experimental import pallas as pl
from jax.experimental.pallas import tpu as pltpu
from jax.experimental.pallas import tpu_sc as plsc

N = 10000
E = 320000
D = 128
KAPPA = 0.95

NC = 2
NS = 16
NW = NC * NS
EPT = E // NW

DCHUNK = 128
DEPAD = 327680
DCPT = DEPAD // (NW * DCHUNK)
DEG_ROWS = 10240
DEG_W = 16
DROWS_PER_TILE = DEG_ROWS // NS
FIRE = 8

CHUNK = 120
FULL = EPT // CHUNK
TAIL = EPT - FULL * CHUNK
GROUPS = FULL // 3
EPI0 = GROUPS * 3
ACC_ROWS = 10112
ROWS_PER_TILE = ACC_ROWS // NS

_mesh = plsc.VectorSubcoreMesh(core_axis_name="c", subcore_axis_name="s")


@functools.partial(
    pl.kernel,
    out_type=jax.ShapeDtypeStruct((NC, DEG_ROWS, DEG_W), jnp.float32),
    mesh=_mesh,
    scratch_types=[
        pltpu.VMEM((DCHUNK, DEG_W), jnp.float32),
        pltpu.VMEM((DCHUNK, DEG_W), jnp.float32),
        pltpu.VMEM((DCPT, DCHUNK), jnp.int32),
        pltpu.VMEM_SHARED((DEG_ROWS, DEG_W), jnp.float32),
        pltpu.SemaphoreType.DMA,
    ],
)
def _sc_degree(dst_hbm, aux_hbm, out_hbm, zero_v, ones_v, idx_v, acc, sem):
    c = lax.axis_index("c")
    s = lax.axis_index("s")
    wid = s * NC + c
    pltpu.sync_copy(aux_hbm.at[0], zero_v)
    pltpu.sync_copy(aux_hbm.at[1], ones_v)
    pltpu.sync_copy(dst_hbm.at[wid], idx_v)
    base = s * DROWS_PER_TILE
    for k in range(DROWS_PER_TILE // DCHUNK):
        pltpu.sync_copy(zero_v, acc.at[pl.ds(base + k * DCHUNK, DCHUNK)])
    plsc.subcore_barrier()

    def body(b, carry):
        j0 = b * FIRE
        for k in range(FIRE):
            pltpu.async_copy(ones_v, acc.at[idx_v.at[j0 + k]], sem, add=True)
        for k in range(FIRE):
            pltpu.make_async_copy(ones_v, acc.at[idx_v.at[j0 + k]], sem).wait()
        return carry

    lax.fori_loop(0, DCPT // FIRE, body, 0)
    plsc.subcore_barrier()
    pltpu.sync_copy(acc.at[pl.ds(base, DROWS_PER_TILE)],
                    out_hbm.at[c, pl.ds(base, DROWS_PER_TILE)])


@functools.partial(
    pl.kernel,
    out_type=jax.ShapeDtypeStruct((NC, ACC_ROWS, D), jnp.float32),
    mesh=_mesh,
    scratch_types=[
        pltpu.VMEM((2, CHUNK), jnp.int32),
        pltpu.VMEM((2, CHUNK), jnp.int32),
        pltpu.VMEM((2, CHUNK), jnp.int32),
        pltpu.VMEM((2, TAIL), jnp.int32),
        pltpu.VMEM((CHUNK, D), jnp.float32),
        pltpu.VMEM((CHUNK, D), jnp.float32),
        pltpu.VMEM((CHUNK, D), jnp.float32),
        pltpu.VMEM_SHARED((ACC_ROWS, D), jnp.float32),
        pltpu.SemaphoreType.DMA,
        pltpu.SemaphoreType.DMA,
        pltpu.SemaphoreType.DMA,
        pltpu.SemaphoreType.DMA,
        pltpu.SemaphoreType.DMA,
        pltpu.SemaphoreType.DMA,
    ],
)
def _sc_aggregate(src_hbm, dst_hbm, sxw_hbm, zeros_hbm, out_hbm,
                  idx0, idx1, idx2, idxt, rows0, rows1, rows2, acc,
                  semg0, semg1, semg2, sems0, sems1, sems2):
    c = lax.axis_index("c")
    s = lax.axis_index("s")
    wid = s * NC + c
    ebase = wid * EPT
    idx = (idx0, idx1, idx2)
    rows = (rows0, rows1, rows2)
    semg = (semg0, semg1, semg2)
    sems = (sems0, sems1, sems2)

    def fetch_idx(slot, chunk):
        off = ebase + chunk * CHUNK
        pltpu.sync_copy(src_hbm.at[pl.ds(off, CHUNK)], idx[slot].at[0])
        pltpu.sync_copy(dst_hbm.at[pl.ds(off, CHUNK)], idx[slot].at[1])

    def gather(slot):
        pltpu.async_copy(sxw_hbm.at[idx[slot].at[0]], rows[slot], semg[slot])

    def wait_gather(slot):
        pltpu.make_async_copy(sxw_hbm.at[idx[slot].at[0]], rows[slot],
                              semg[slot]).wait()

    def scatter(slot):
        pltpu.async_copy(rows[slot], acc.at[idx[slot].at[1]], sems[slot],
                         add=True)

    def wait_scatter(slot):
        pltpu.make_async_copy(rows[slot], acc.at[idx[slot].at[1]],
                              sems[slot]).wait()

    for k in range(3):
        fetch_idx(k, k)
    pltpu.sync_copy(zeros_hbm, rows2)
    base = s * ROWS_PER_TILE
    for k in range(ROWS_PER_TILE // CHUNK):
        pltpu.sync_copy(rows2.at[pl.ds(0, CHUNK)],
                        acc.at[pl.ds(base + k * CHUNK, CHUNK)])
    rem = ROWS_PER_TILE - (ROWS_PER_TILE // CHUNK) * CHUNK
    pltpu.sync_copy(rows2.at[pl.ds(0, rem)],
                    acc.at[pl.ds(base + ROWS_PER_TILE - rem, rem)])
    for k in range(3):
        gather(k)
    plsc.subcore_barrier()

    def body(g, carry):
        for k in range(3):
            wait_gather(k)
            scatter(k)
        not_last = g + 1 < GROUPS

        @pl.when(not_last)
        def _():
            for k in range(3):
                wait_scatter(k)
                fetch_idx(k, 3 * (g + 1) + k)
                gather(k)

        return carry

    lax.fori_loop(0, GROUPS, body, 0)

    for k in range(2):
        wait_scatter(k)
        fetch_idx(k, EPI0 + k)
        gather(k)
    wait_scatter(2)
    toff = ebase + FULL * CHUNK
    pltpu.sync_copy(src_hbm.at[pl.ds(toff, TAIL)], idxt.at[0])
    pltpu.sync_copy(dst_hbm.at[pl.ds(toff, TAIL)], idxt.at[1])
    pltpu.async_copy(sxw_hbm.at[idxt.at[0]], rows2.at[pl.ds(0, TAIL)], semg2)
    for k in range(2):
        wait_gather(k)
        scatter(k)
    pltpu.make_async_copy(sxw_hbm.at[idxt.at[0]], rows2.at[pl.ds(0, TAIL)],
                          semg2).wait()
    pltpu.async_copy(rows2.at[pl.ds(0, TAIL)], acc.at[idxt.at[1]], sems2,
                     add=True)
    for k in range(2):
        wait_scatter(k)
    pltpu.make_async_copy(rows2.at[pl.ds(0, TAIL)], acc.at[idxt.at[1]],
                          sems2).wait()

    plsc.subcore_barrier()
    pltpu.sync_copy(acc.at[pl.ds(base, ROWS_PER_TILE)],
                    out_hbm.at[c, pl.ds(base, ROWS_PER_TILE)])


def _project(W):
    Wabs = jnp.abs(W)
    row_sum = jnp.sum(Wabs, axis=1, keepdims=True)
    hi0 = jnp.max(Wabs, axis=1, keepdims=True)

    def bis(i, lohi):
        lo, hi = lohi
        mid = 0.5 * (lo + hi)
        g = jnp.sum(jnp.maximum(Wabs - mid, 0.0), axis=1, keepdims=True)
        gt = g > KAPPA
        return (jnp.where(gt, mid, lo), jnp.where(gt, hi, mid))

    lo, hi = lax.fori_loop(0, 40, bis, (jnp.zeros_like(hi0), hi0))
    theta = 0.5 * (lo + hi)
    proj = jnp.sign(W) * jnp.maximum(Wabs - theta, 0.0)
    return jnp.where(row_sum > KAPPA, proj, W)


def _dinv_from_parts(degp_ref):
    deg = degp_ref[0, 0:N, 0:1] + degp_ref[1, 0:N, 0:1] + 1.0
    return lax.rsqrt(deg)


def _tc_xw_body(emb_ref, w_ref, out_ref):
    Wc = _project(w_ref[...])
    out_ref[...] = lax.dot_general(emb_ref[...], Wc, (((1,), (1,)), ((), ())),
                                   preferred_element_type=jnp.float32)


def _tc_scale_body(xw_ref, degp_ref, out_ref):
    out_ref[...] = xw_ref[...] * _dinv_from_parts(degp_ref)


def _tc_finish_body(parts_ref, sxw_ref, feat_ref, wmlp_ref, degp_ref, out_ref):
    dinv = _dinv_from_parts(degp_ref)
    agg = parts_ref[0, 0:N] + parts_ref[1, 0:N] + sxw_ref[...]
    mlp = lax.dot_general(feat_ref[...], wmlp_ref[...],
                          (((1,), (1,)), ((), ())),
                          preferred_element_type=jnp.float32)
    out_ref[...] = jnp.maximum(agg * dinv + mlp, 0.0)


def kernel(features, sparse_adj, embeddings, W_conv, W_mlp):
    dst = sparse_adj[1]
    pad = DEPAD - E
    pad_idx = jnp.arange(pad, dtype=jnp.int32)
    dst_p = jnp.concatenate([dst, N + pad_idx % (DEG_ROWS - N)])
    dst3 = dst_p.reshape(NW, DCPT, DCHUNK)

    aux16 = jnp.stack([jnp.zeros((DCHUNK, DEG_W), jnp.float32),
                       jnp.ones((DCHUNK, DEG_W), jnp.float32)])
    zeros128 = jnp.zeros((CHUNK, D), jnp.float32)

    degp = _sc_degree(dst3, aux16)

    xw = pl.pallas_call(
        _tc_xw_body,
        out_shape=jax.ShapeDtypeStruct((N, D), jnp.float32),
    )(embeddings, W_conv)

    sxw = pl.pallas_call(
        _tc_scale_body,
        out_shape=jax.ShapeDtypeStruct((N, D), jnp.float32),
    )(xw, degp)

    parts = _sc_aggregate(sparse_adj[0], sparse_adj[1], sxw, zeros128)

    y = pl.pallas_call(
        _tc_finish_body,
        out_shape=jax.ShapeDtypeStruct((N, D), jnp.float32),
    )(parts, sxw, features, W_mlp, degp)
    return y


# speedup vs baseline: 1.0800x; 1.0800x over previous
import functools

import jax
import jax.numpy as jnp
from jax import lax
from jax.experimental import pallas as pl
from jax.experimental.pallas import tpu as pltpu
from jax.experimental.pallas import tpu_sc as plsc

N = 10000
E = 320000
D = 128
KAPPA = 0.95

NC = 2
NS = 16
NW = NC * NS
CHUNK = 128
EPAD = 327680
CPT = EPAD // (NW * CHUNK)
ACC_ROWS = 10240
DUMMY = N
DEG_W = 16
ROWS_PER_TILE = ACC_ROWS // NS
FIRE = 8
NCHUNKS = E // CHUNK
ACPT = NCHUNKS // NW
AEXTRA = NCHUNKS - ACPT * NW

_mesh = plsc.VectorSubcoreMesh(core_axis_name="c", subcore_axis_name="s")


@functools.partial(
    pl.kernel,
    out_type=jax.ShapeDtypeStruct((NC, ACC_ROWS, DEG_W), jnp.float32),
    mesh=_mesh,
    scratch_types=[
        pltpu.VMEM((CHUNK, DEG_W), jnp.float32),
        pltpu.VMEM((CHUNK, DEG_W), jnp.float32),
        pltpu.VMEM((CPT, CHUNK), jnp.int32),
        pltpu.VMEM_SHARED((ACC_ROWS, DEG_W), jnp.float32),
        pltpu.SemaphoreType.DMA,
    ],
)
def _sc_degree(dst_hbm, aux_hbm, out_hbm, zero_v, ones_v, idx_v, acc, sem):
    c = lax.axis_index("c")
    s = lax.axis_index("s")
    wid = s * NC + c
    pltpu.sync_copy(aux_hbm.at[0], zero_v)
    pltpu.sync_copy(aux_hbm.at[1], ones_v)
    pltpu.sync_copy(dst_hbm.at[wid], idx_v)
    base = s * ROWS_PER_TILE
    for k in range(ROWS_PER_TILE // CHUNK):
        pltpu.sync_copy(zero_v, acc.at[pl.ds(base + k * CHUNK, CHUNK)])
    plsc.subcore_barrier()

    def body(b, carry):
        j0 = b * FIRE
        for k in range(FIRE):
            pltpu.async_copy(ones_v, acc.at[idx_v.at[j0 + k]], sem, add=True)
        for k in range(FIRE):
            pltpu.make_async_copy(ones_v, acc.at[idx_v.at[j0 + k]], sem).wait()
        return carry

    lax.fori_loop(0, CPT // FIRE, body, 0)
    plsc.subcore_barrier()
    pltpu.sync_copy(acc.at[pl.ds(base, ROWS_PER_TILE)],
                    out_hbm.at[c, pl.ds(base, ROWS_PER_TILE)])


@functools.partial(
    pl.kernel,
    out_type=jax.ShapeDtypeStruct((NC, ACC_ROWS, D), jnp.float32),
    mesh=_mesh,
    scratch_types=[
        pltpu.VMEM((2, CHUNK), jnp.int32),
        pltpu.VMEM((2, CHUNK), jnp.int32),
        pltpu.VMEM((CHUNK, D), jnp.float32),
        pltpu.VMEM((CHUNK, D), jnp.float32),
        pltpu.VMEM_SHARED((ACC_ROWS, D), jnp.float32),
        pltpu.SemaphoreType.DMA,
        pltpu.SemaphoreType.DMA,
        pltpu.SemaphoreType.DMA,
    ],
)
def _sc_aggregate(edges_hbm, sxw_hbm, zeros_hbm, out_hbm,
                  idx0, idx1, rows0, rows1, acc, sem0, sem1, semi):
    c = lax.axis_index("c")
    s = lax.axis_index("s")
    wid = s * NC + c
    start = wid * ACPT + jnp.minimum(wid, AEXTRA)
    npairs = ACPT // 2
    pltpu.sync_copy(edges_hbm.at[start], idx0)
    pltpu.async_copy(edges_hbm.at[start + 1], idx1, semi)
    pltpu.async_copy(sxw_hbm.at[idx0.at[0]], rows0, sem0)
    pltpu.sync_copy(zeros_hbm, rows1)
    base = s * ROWS_PER_TILE
    for k in range(ROWS_PER_TILE // CHUNK):
        pltpu.sync_copy(rows1, acc.at[pl.ds(base + k * CHUNK, CHUNK)])
    plsc.subcore_barrier()

    def body(t, carry):
        j0 = start + 2 * t
        not_last = t + 1 < npairs
        pltpu.make_async_copy(edges_hbm.at[j0 + 1], idx1, semi).wait()
        pltpu.async_copy(sxw_hbm.at[idx1.at[0]], rows1, sem1)
        pltpu.make_async_copy(sxw_hbm.at[idx0.at[0]], rows0, sem0).wait()
        pltpu.sync_copy(rows0, acc.at[idx0.at[1]], add=True)

        @pl.when(not_last)
        def _():
            pltpu.sync_copy(edges_hbm.at[j0 + 2], idx0)
            pltpu.async_copy(sxw_hbm.at[idx0.at[0]], rows0, sem0)

        pltpu.make_async_copy(sxw_hbm.at[idx1.at[0]], rows1, sem1).wait()
        pltpu.sync_copy(rows1, acc.at[idx1.at[1]], add=True)

        @pl.when(not_last)
        def _():
            pltpu.async_copy(edges_hbm.at[j0 + 3], idx1, semi)

        return carry

    lax.fori_loop(0, npairs, body, 0)

    @pl.when(wid < AEXTRA)
    def _():
        pltpu.sync_copy(edges_hbm.at[start + ACPT], idx0)
        pltpu.async_copy(sxw_hbm.at[idx0.at[0]], rows0, sem0)
        pltpu.make_async_copy(sxw_hbm.at[idx0.at[0]], rows0, sem0).wait()
        pltpu.sync_copy(rows0, acc.at[idx0.at[1]], add=True)

    plsc.subcore_barrier()
    pltpu.sync_copy(acc.at[pl.ds(base, ROWS_PER_TILE)],
                    out_hbm.at[c, pl.ds(base, ROWS_PER_TILE)])


def _project(W):
    Wabs = jnp.abs(W)
    row_sum = jnp.sum(Wabs, axis=1, keepdims=True)
    hi0 = jnp.max(Wabs, axis=1, keepdims=True)

    def bis(i, lohi):
        lo, hi = lohi
        mid = 0.5 * (lo + hi)
        g = jnp.sum(jnp.maximum(Wabs - mid, 0.0), axis=1, keepdims=True)
        gt = g > KAPPA
        return (jnp.where(gt, mid, lo), jnp.where(gt, hi, mid))

    lo, hi = lax.fori_loop(0, 40, bis, (jnp.zeros_like(hi0), hi0))
    theta = 0.5 * (lo + hi)
    proj = jnp.sign(W) * jnp.maximum(Wabs - theta, 0.0)
    return jnp.where(row_sum > KAPPA, proj, W)


def _dinv_from_parts(degp_ref):
    deg = degp_ref[0, 0:N, 0:1] + degp_ref[1, 0:N, 0:1] + 1.0
    return lax.rsqrt(deg)


def _tc_xw_body(emb_ref, w_ref, out_ref):
    Wc = _project(w_ref[...])
    out_ref[...] = lax.dot_general(emb_ref[...], Wc, (((1,), (1,)), ((), ())),
                                   preferred_element_type=jnp.float32)


def _tc_scale_body(xw_ref, degp_ref, out_ref):
    out_ref[...] = xw_ref[...] * _dinv_from_parts(degp_ref)


def _tc_finish_body(parts_ref, sxw_ref, feat_ref, wmlp_ref, degp_ref, out_ref):
    dinv = _dinv_from_parts(degp_ref)
    agg = parts_ref[0, 0:N] + parts_ref[1, 0:N] + sxw_ref[...]
    mlp = lax.dot_general(feat_ref[...], wmlp_ref[...],
                          (((1,), (1,)), ((), ())),
                          preferred_element_type=jnp.float32)
    out_ref[...] = jnp.maximum(agg * dinv + mlp, 0.0)


def kernel(features, sparse_adj, embeddings, W_conv, W_mlp):
    dst = sparse_adj[1]
    pad = EPAD - E
    pad_idx = jnp.arange(pad, dtype=jnp.int32)
    dst_p = jnp.concatenate([dst, N + pad_idx % (ACC_ROWS - N)])
    edges = sparse_adj.reshape(2, NCHUNKS, CHUNK).transpose(1, 0, 2)

    aux16 = jnp.stack([jnp.zeros((CHUNK, DEG_W), jnp.float32),
                       jnp.ones((CHUNK, DEG_W), jnp.float32)])
    zeros128 = jnp.zeros((CHUNK, D), jnp.float32)

    degp = _sc_degree(dst_p.reshape(NW, CPT, CHUNK), aux16)

    xw = pl.pallas_call(
        _tc_xw_body,
        out_shape=jax.ShapeDtypeStruct((N, D), jnp.float32),
    )(embeddings, W_conv)

    sxw = pl.pallas_call(
        _tc_scale_body,
        out_shape=jax.ShapeDtypeStruct((N, D), jnp.float32),
    )(xw, degp)

    parts = _sc_aggregate(edges, sxw, zeros128)

    y = pl.pallas_call(
        _tc_finish_body,
        out_shape=jax.ShapeDtypeStruct((N, D), jnp.float32),
    )(parts, sxw, features, W_mlp, degp)
    return y
